# baseline (speedup 1.0000x reference)
import functools

import jax
import jax.numpy as jnp
from jax import lax
from jax.experimental import pallas as pl
from jax.experimental.pallas import tpu as pltpu
from jax.experimental.pallas import tpu_sc as plsc

_B, _S = 16, 2048
_NW = 32
_CHUNK = (_B * _S) // _NW
_L = 16


def _sc_body(scores_hbm, masks_hbm, bias_hbm, out_hbm,
             scores_v, masks_v, out_v, bias_v):
    wid = lax.axis_index("s") * 2 + lax.axis_index("c")
    base = wid * _CHUNK
    pltpu.sync_copy(bias_hbm, bias_v.at[pl.ds(0, 1)])
    pltpu.sync_copy(scores_hbm.at[pl.ds(base, _CHUNK)], scores_v)
    pltpu.sync_copy(masks_hbm.at[pl.ds(base, _CHUNK)], masks_v)
    b = bias_v[...][0]
    for i in range(_CHUNK // _L):
        sl = pl.ds(i * _L, _L)
        out_v[sl] = (scores_v[sl] + b) * masks_v[sl].astype(jnp.float32)
    pltpu.sync_copy(out_v, out_hbm.at[pl.ds(base, _CHUNK)])


@functools.partial(jax.jit, static_argnums=())
def _sc_call(scores_flat, masks_flat, bias_vec):
    mesh = plsc.VectorSubcoreMesh(core_axis_name="c", subcore_axis_name="s")
    fn = pl.kernel(
        _sc_body,
        out_type=jax.ShapeDtypeStruct((_B * _S,), jnp.float32),
        mesh=mesh,
        scratch_types=[
            pltpu.VMEM((_CHUNK,), jnp.float32),
            pltpu.VMEM((_CHUNK,), jnp.int32),
            pltpu.VMEM((_CHUNK,), jnp.float32),
            pltpu.VMEM((_L,), jnp.float32),
        ],
    )
    return fn(scores_flat, masks_flat, bias_vec)


def kernel(sent_group_scores, sel_sent_emb, sel_sent_masks, group_embs, candi_sent_masks, bias):
    del sel_sent_emb, sel_sent_masks, group_embs
    out = _sc_call(
        sent_group_scores.reshape(_B * _S),
        candi_sent_masks.reshape(_B * _S),
        bias.reshape(1),
    )
    return out.reshape(_B, _S)


# speedup vs baseline: 9.9122x; 9.9122x over previous
import jax
import jax.numpy as jnp
from jax.experimental import pallas as pl
from jax.experimental.pallas import tpu as pltpu


def _ew_kernel(bias_ref, scores_ref, masks_ref, out_ref):
    out_ref[...] = (scores_ref[...] + bias_ref[()]) * masks_ref[...].astype(jnp.float32)


def kernel(sent_group_scores, sel_sent_emb, sel_sent_masks, group_embs, candi_sent_masks, bias):
    del sel_sent_emb, sel_sent_masks, group_embs
    return pl.pallas_call(
        _ew_kernel,
        in_specs=[
            pl.BlockSpec(memory_space=pltpu.SMEM),
            pl.BlockSpec(memory_space=pltpu.VMEM),
            pl.BlockSpec(memory_space=pltpu.VMEM),
        ],
        out_specs=pl.BlockSpec(memory_space=pltpu.VMEM),
        out_shape=jax.ShapeDtypeStruct(sent_group_scores.shape, jnp.float32),
    )(bias, sent_group_scores, candi_sent_masks)
